# Initial kernel scaffold; baseline (speedup 1.0000x reference)
import jax
import jax.numpy as jnp
from jax.experimental import pallas as pl


def kernel(indices, embeddings):
    raise NotImplementedError("write your pallas kernel here")


# speedup vs baseline: 1.0939x; 1.0939x over previous
import functools

import jax
import jax.numpy as jnp
from jax import lax
from jax.experimental import pallas as pl
from jax.experimental.pallas import tpu as pltpu
from jax.experimental.pallas import tpu_sc as plsc

B, S = 16384, 50
D = 32
N = B * S
NC, NS = 2, 16
NW = NC * NS
PER_W = N // NW
K = 8
CHUNK = K * 128
NCHUNK = PER_W // CHUNK
ROWS_PER_W = PER_W // 128


def _gather_kernel(idx_hbm, table_hbm, out_hbm, idx_v, rows_v, sem):
    wid = lax.axis_index("s") * NC + lax.axis_index("c")
    row_base = wid * ROWS_PER_W
    out_base = wid * PER_W

    def chunk_body(c, carry):
        crow = row_base + c * K
        pltpu.sync_copy(idx_hbm.at[pl.ds(crow, K)], idx_v)
        copies = []
        for j in range(K):
            copies.append(
                pltpu.async_copy(
                    table_hbm.at[idx_v.at[j]],
                    rows_v.at[pl.ds(j * 128, 128)],
                    sem,
                )
            )
        for cp in copies:
            cp.wait()
        pltpu.sync_copy(rows_v, out_hbm.at[pl.ds(out_base + c * CHUNK, CHUNK)])
        return carry

    lax.fori_loop(0, NCHUNK, chunk_body, 0)


@jax.jit
def _run(idx2d, table):
    mesh = plsc.VectorSubcoreMesh(core_axis_name="c", subcore_axis_name="s")
    f = functools.partial(
        pl.kernel,
        mesh=mesh,
        out_type=jax.ShapeDtypeStruct((N, D), jnp.float32),
        scratch_types=[
            pltpu.VMEM((K, 128), jnp.int32),
            pltpu.VMEM((CHUNK, D), jnp.float32),
            pltpu.SemaphoreType.DMA,
        ],
        compiler_params=pltpu.CompilerParams(use_tc_tiling_on_sc=False),
    )(_gather_kernel)
    return f(idx2d, table)


def kernel(indices, embeddings):
    idx2d = indices.astype(jnp.int32).reshape(N // 128, 128)
    out = _run(idx2d, embeddings)
    return out.reshape(B, S, D)


# speedup vs baseline: 1.0994x; 1.0050x over previous
import functools

import jax
import jax.numpy as jnp
from jax import lax
from jax.experimental import pallas as pl
from jax.experimental.pallas import tpu as pltpu
from jax.experimental.pallas import tpu_sc as plsc

B, S = 16384, 50
D = 32
N = B * S
NC, NS = 2, 16
NW = NC * NS
PER_W = N // NW
K = 8
CHUNK = K * 128
NCHUNK = PER_W // CHUNK
ROWS_PER_W = PER_W // 128
NBUF = 3


def _gather_kernel(idx_hbm, table_hbm, out_hbm, idx_v, rows_v, sem_g, sem_o):
    wid = lax.axis_index("s") * NC + lax.axis_index("c")
    row_base = wid * ROWS_PER_W
    out_base = wid * PER_W

    pltpu.sync_copy(idx_hbm.at[pl.ds(row_base, ROWS_PER_W)], idx_v)

    def fire_gathers(c, b):
        for j in range(K):
            pltpu.async_copy(
                table_hbm.at[idx_v.at[c * K + j]],
                rows_v.at[pl.ds(b * CHUNK + j * 128, 128)],
                sem_g.at[b],
            )

    def wait_gathers(b):
        pltpu.make_async_copy(
            out_hbm.at[pl.ds(0, CHUNK)],
            rows_v.at[pl.ds(b * CHUNK, CHUNK)],
            sem_g.at[b],
        ).wait()

    def fire_out(c, b):
        pltpu.async_copy(
            rows_v.at[pl.ds(b * CHUNK, CHUNK)],
            out_hbm.at[pl.ds(out_base + c * CHUNK, CHUNK)],
            sem_o.at[b],
        )

    def wait_out(b):
        pltpu.make_async_copy(
            out_hbm.at[pl.ds(0, CHUNK)],
            rows_v.at[pl.ds(b * CHUNK, CHUNK)],
            sem_o.at[b],
        ).wait()

    for c in range(NBUF - 1):
        fire_gathers(c, c)

    def loop_body(c, carry):
        b = lax.rem(c, NBUF)

        @pl.when(c >= NBUF)
        def _():
            wait_out(b)

        fire_gathers(c, b)
        co = c - (NBUF - 1)
        b2 = lax.rem(co, NBUF)
        wait_gathers(b2)
        fire_out(co, b2)
        return carry

    lax.fori_loop(NBUF - 1, NCHUNK, loop_body, 0)

    for co in range(NCHUNK - (NBUF - 1), NCHUNK):
        b2 = co % NBUF
        wait_gathers(b2)
        fire_out(co, b2)
    for b in range(NBUF):
        wait_out(b)


@jax.jit
def _run(idx2d, table):
    mesh = plsc.VectorSubcoreMesh(core_axis_name="c", subcore_axis_name="s")
    f = functools.partial(
        pl.kernel,
        mesh=mesh,
        out_type=jax.ShapeDtypeStruct((N, D), jnp.float32),
        scratch_types=[
            pltpu.VMEM((ROWS_PER_W, 128), jnp.int32),
            pltpu.VMEM((NBUF * CHUNK, D), jnp.float32),
            pltpu.SemaphoreType.DMA((NBUF,)),
            pltpu.SemaphoreType.DMA((NBUF,)),
        ],
        compiler_params=pltpu.CompilerParams(use_tc_tiling_on_sc=False),
    )(_gather_kernel)
    return f(idx2d, table)


def kernel(indices, embeddings):
    idx2d = indices.astype(jnp.int32).reshape(N // 128, 128)
    out = _run(idx2d, embeddings)
    return out.reshape(B, S, D)


# speedup vs baseline: 1.4952x; 1.3601x over previous
import functools

import jax
import jax.numpy as jnp
from jax import lax
from jax.experimental import pallas as pl
from jax.experimental.pallas import tpu as pltpu
from jax.experimental.pallas import tpu_sc as plsc

B, S = 16384, 50
D = 32
V = 1000000
NC, NS = 2, 16
NW = NC * NS
NB_PER_W = B // NW
BCH = 128
NJ = NB_PER_W // BCH
NIT = S * NJ
NSLOT = 6
PRO = NSLOT - 1


def _transpose_block(rows, tp):
    for h in range(8):
        row_ids = jnp.arange(16, dtype=jnp.int32) + (h * 16)
        for r in range(D):
            col_ids = jnp.full((16,), r, dtype=jnp.int32)
            v = plsc.load_gather(rows, [row_ids, col_ids])
            tp[r, pl.ds(h * 16, 16)] = v


def _gather_kernel(idxT_hbm, table_hbm, out_hbm, idx_v, rows_v, tp_v,
                   sem_i, sem_g, sem_o):
    wid = lax.axis_index("s") * NC + lax.axis_index("c")
    b_base = wid * NB_PER_W

    idx_copies = []
    for s in range(S):
        idx_copies.append(
            pltpu.async_copy(
                idxT_hbm.at[s, pl.ds(b_base, NB_PER_W)],
                idx_v.at[s],
                sem_i,
            )
        )
    for cp in idx_copies:
        cp.wait()

    def fire_gather(i, slot):
        s = i // NJ
        j = i - s * NJ
        pltpu.async_copy(
            table_hbm.at[idx_v.at[s, pl.ds(j * BCH, BCH)]],
            rows_v.at[pl.ds(slot * BCH, BCH)],
            sem_g.at[slot],
        )

    def wait_gather(slot):
        pltpu.make_async_copy(
            table_hbm.at[pl.ds(0, BCH)],
            rows_v.at[pl.ds(slot * BCH, BCH)],
            sem_g.at[slot],
        ).wait()

    def fire_write(i, slot):
        s = i // NJ
        j = i - s * NJ
        pltpu.async_copy(
            tp_v.at[pl.ds(slot * D, D)],
            out_hbm.at[s, :, pl.ds(b_base + j * BCH, BCH)],
            sem_o.at[slot],
        )

    def wait_write(slot):
        pltpu.make_async_copy(
            tp_v.at[pl.ds(slot * D, D)],
            out_hbm.at[0, :, pl.ds(0, BCH)],
            sem_o.at[slot],
        ).wait()

    for i in range(PRO):
        fire_gather(i, i % NSLOT)

    def body(i, carry):
        slot = lax.rem(i, NSLOT)

        @pl.when(i >= NSLOT)
        def _():
            wait_write(slot)

        wait_gather(slot)
        _transpose_block(
            rows_v.at[pl.ds(slot * BCH, BCH)],
            tp_v.at[pl.ds(slot * D, D)],
        )
        fire_write(i, slot)
        ip = i + PRO

        @pl.when(ip < NIT)
        def _():
            fire_gather(ip, lax.rem(ip, NSLOT))

        return carry

    lax.fori_loop(0, NIT, body, 0)

    for slot in range(NSLOT):
        wait_write(slot)


@jax.jit
def _run(idxT, table):
    mesh = plsc.VectorSubcoreMesh(core_axis_name="c", subcore_axis_name="s")
    f = functools.partial(
        pl.kernel,
        mesh=mesh,
        out_type=jax.ShapeDtypeStruct((S, D, B), jnp.float32),
        scratch_types=[
            pltpu.VMEM((S, NB_PER_W), jnp.int32),
            pltpu.VMEM((NSLOT * BCH, D), jnp.float32),
            pltpu.VMEM((NSLOT * D, BCH), jnp.float32),
            pltpu.SemaphoreType.DMA,
            pltpu.SemaphoreType.DMA((NSLOT,)),
            pltpu.SemaphoreType.DMA((NSLOT,)),
        ],
        compiler_params=pltpu.CompilerParams(
            use_tc_tiling_on_sc=False, needs_layout_passes=False
        ),
    )(_gather_kernel)
    return f(idxT, table)


def kernel(indices, embeddings):
    idxT = indices.astype(jnp.int32).T
    w3 = _run(idxT, embeddings)
    return jnp.transpose(w3, (2, 0, 1))
